# Initial kernel scaffold; baseline (speedup 1.0000x reference)
import jax
import jax.numpy as jnp
from jax.experimental import pallas as pl


def kernel(skill_weights, skill_embeddings, W_ih, W_hh, b_ih, b_hh, gp_w1, gp_b1, gp_g1, sw_w1, sw_w3, sw_w2, gp_w2, gp_b2, gp_g2):
    raise NotImplementedError("write your pallas kernel here")


# speedup vs baseline: 1.4649x; 1.4649x over previous
import jax
import jax.numpy as jnp
from jax.experimental import pallas as pl
from jax.experimental.pallas import tpu as pltpu

_B_BLK = 1024
_C_BLK = 512
_N_COMPOSE = 3
_HIDDEN = 256
_EPS = 1e-6


def _kern(sw_ref, emb_ref, wih_ref, whh_ref, bih_ref, bhh_ref,
          w1_ref, b1_ref, g1_ref, sww1_ref, sww3_ref, sww2_ref,
          w2_ref, b2_ref, g2_ref, out_ref, x_scr, acc_scr):
    c = pl.program_id(1)
    n_c = pl.num_programs(1)

    @pl.when(c == 0)
    def _routing_gru_proj():
        sw = sw_ref[...]
        n_slots = sw.shape[1]
        iota = jax.lax.broadcasted_iota(jnp.int32, sw.shape, 1)
        vals, onehots = [], []
        cur = sw
        for _ in range(_N_COMPOSE):
            m = jnp.max(cur, axis=1, keepdims=True)
            idx = jnp.min(jnp.where(cur == m, iota, n_slots), axis=1,
                          keepdims=True)
            oh = iota == idx
            vals.append(m)
            onehots.append(oh)
            cur = jnp.where(oh, -jnp.inf, cur)
        es = [jnp.exp(v - vals[0]) for v in vals]
        denom = es[0] + es[1] + es[2]
        emb = emb_ref[...]
        h = jnp.zeros((sw.shape[0], _HIDDEN), jnp.float32)
        wih = wih_ref[...]
        whh = whh_ref[...]
        bih = bih_ref[...]
        bhh = bhh_ref[...]
        for t in range(_N_COMPOSE):
            mix = es[t] / denom
            w_oh = jnp.where(onehots[t], mix, 0.0).astype(jnp.bfloat16)
            x_t = jnp.dot(w_oh, emb, preferred_element_type=jnp.float32)
            gi = jnp.dot(x_t.astype(jnp.bfloat16), wih,
                         preferred_element_type=jnp.float32) + bih
            gh = jnp.dot(h.astype(jnp.bfloat16), whh,
                         preferred_element_type=jnp.float32) + bhh
            i_r, i_z, i_n = jnp.split(gi, 3, axis=-1)
            h_r, h_z, h_n = jnp.split(gh, 3, axis=-1)
            r = jax.nn.sigmoid(i_r + h_r)
            z = jax.nn.sigmoid(i_z + h_z)
            n = jnp.tanh(i_n + r * h_n)
            h = (1.0 - z) * n + z * h
        x = jnp.dot(h.astype(jnp.bfloat16), w1_ref[...],
                    preferred_element_type=jnp.float32) + b1_ref[...]
        x = x * jax.lax.rsqrt(
            jnp.mean(x * x, axis=-1, keepdims=True) + _EPS) * g1_ref[...]
        x_scr[...] = x.astype(jnp.bfloat16)

    xb = x_scr[...]
    a = jnp.dot(xb, sww1_ref[...], preferred_element_type=jnp.float32)
    b = jnp.dot(xb, sww3_ref[...], preferred_element_type=jnp.float32)
    mid = (jax.nn.silu(a) * b).astype(jnp.bfloat16)
    contrib = jnp.dot(mid, sww2_ref[...], preferred_element_type=jnp.float32)

    @pl.when(c == 0)
    def _init_acc():
        acc_scr[...] = contrib

    @pl.when(c > 0)
    def _accum():
        acc_scr[...] += contrib

    @pl.when(c == n_c - 1)
    def _finish():
        y = jnp.dot(acc_scr[...].astype(jnp.bfloat16), w2_ref[...],
                    preferred_element_type=jnp.float32) + b2_ref[...]
        y = y * jax.lax.rsqrt(
            jnp.mean(y * y, axis=-1, keepdims=True) + _EPS) * g2_ref[...]
        out_ref[...] = y


def kernel(skill_weights, skill_embeddings, W_ih, W_hh, b_ih, b_hh,
           gp_w1, gp_b1, gp_g1, sw_w1, sw_w3, sw_w2, gp_w2, gp_b2, gp_g2):
    batch, n_slots = skill_weights.shape
    embed = skill_embeddings.shape[1]
    dim = gp_w1.shape[1]
    inner = sw_w1.shape[1]
    goal = gp_w2.shape[1]
    n_b = batch // _B_BLK
    n_c = inner // _C_BLK

    bf = jnp.bfloat16
    emb = skill_embeddings.astype(bf)
    wih_t = W_ih.T.astype(bf)
    whh_t = W_hh.T.astype(bf)
    bih = b_ih.reshape(1, -1)
    bhh = b_hh.reshape(1, -1)
    w1 = gp_w1.astype(bf)
    b1 = gp_b1.reshape(1, -1)
    g1 = gp_g1.reshape(1, -1)
    sw1 = sw_w1.astype(bf)
    sw3 = sw_w3.astype(bf)
    sw2 = sw_w2.astype(bf)
    w2 = gp_w2.astype(bf)
    b2 = gp_b2.reshape(1, -1)
    g2 = gp_g2.reshape(1, -1)

    three_h = wih_t.shape[1]

    grid = (n_b, n_c)
    const = lambda i, c: (0, 0)
    out = pl.pallas_call(
        _kern,
        grid=grid,
        in_specs=[
            pl.BlockSpec((_B_BLK, n_slots), lambda i, c: (i, 0)),
            pl.BlockSpec((n_slots, embed), const),
            pl.BlockSpec((embed, three_h), const),
            pl.BlockSpec((_HIDDEN, three_h), const),
            pl.BlockSpec((1, three_h), const),
            pl.BlockSpec((1, three_h), const),
            pl.BlockSpec((_HIDDEN, dim), const),
            pl.BlockSpec((1, dim), const),
            pl.BlockSpec((1, dim), const),
            pl.BlockSpec((dim, _C_BLK), lambda i, c: (0, c)),
            pl.BlockSpec((dim, _C_BLK), lambda i, c: (0, c)),
            pl.BlockSpec((_C_BLK, dim), lambda i, c: (c, 0)),
            pl.BlockSpec((dim, goal), const),
            pl.BlockSpec((1, goal), const),
            pl.BlockSpec((1, goal), const),
        ],
        out_specs=pl.BlockSpec((_B_BLK, goal), lambda i, c: (i, 0)),
        out_shape=jax.ShapeDtypeStruct((batch, goal), jnp.float32),
        scratch_shapes=[
            pltpu.VMEM((_B_BLK, dim), jnp.bfloat16),
            pltpu.VMEM((_B_BLK, dim), jnp.float32),
        ],
        compiler_params=pltpu.CompilerParams(
            dimension_semantics=("arbitrary", "arbitrary"),
        ),
    )(skill_weights, emb, wih_t, whh_t, bih, bhh,
      w1, b1, g1, sw1, sw3, sw2, w2, b2, g2)
    return out


# speedup vs baseline: 1.7291x; 1.1803x over previous
import jax
import jax.numpy as jnp
from jax.experimental import pallas as pl
from jax.experimental.pallas import tpu as pltpu

_B_BLK = 1024
_C_BLK = 512
_N_COMPOSE = 3
_HIDDEN = 256
_EPS = 1e-6


def _fuse_kern(a_ref, b_ref, out_ref):
    out_ref[...] = jnp.dot(a_ref[...], b_ref[...],
                           preferred_element_type=jnp.float32
                           ).astype(jnp.bfloat16)


def _kern(sw_ref, emb_ref, wih_ref, whh_ref, bih_ref, bhh_ref,
          w1_ref, b1_ref, g1_ref, sww1_ref, sww3_ref, fw_ref,
          b2_ref, g2_ref, out_ref, x_scr, acc_scr):
    c = pl.program_id(1)
    n_c = pl.num_programs(1)

    @pl.when(c == 0)
    def _routing_gru_proj():
        sw = sw_ref[...]
        n_slots = sw.shape[1]
        iota = jax.lax.broadcasted_iota(jnp.int32, sw.shape, 1)
        vals, onehots = [], []
        cur = sw
        for _ in range(_N_COMPOSE):
            m = jnp.max(cur, axis=1, keepdims=True)
            idx = jnp.min(jnp.where(cur == m, iota, n_slots), axis=1,
                          keepdims=True)
            oh = iota == idx
            vals.append(m)
            onehots.append(oh)
            cur = jnp.where(oh, -jnp.inf, cur)
        es = [jnp.exp(v - vals[0]) for v in vals]
        denom = es[0] + es[1] + es[2]
        emb = emb_ref[...]
        h = jnp.zeros((sw.shape[0], _HIDDEN), jnp.float32)
        wih = wih_ref[...]
        whh = whh_ref[...]
        bih = bih_ref[...]
        bhh = bhh_ref[...]
        for t in range(_N_COMPOSE):
            mix = es[t] / denom
            w_oh = jnp.where(onehots[t], mix, 0.0).astype(jnp.bfloat16)
            x_t = jnp.dot(w_oh, emb, preferred_element_type=jnp.float32)
            gi = jnp.dot(x_t.astype(jnp.bfloat16), wih,
                         preferred_element_type=jnp.float32) + bih
            gh = jnp.dot(h.astype(jnp.bfloat16), whh,
                         preferred_element_type=jnp.float32) + bhh
            i_r, i_z, i_n = jnp.split(gi, 3, axis=-1)
            h_r, h_z, h_n = jnp.split(gh, 3, axis=-1)
            r = jax.nn.sigmoid(i_r + h_r)
            z = jax.nn.sigmoid(i_z + h_z)
            n = jnp.tanh(i_n + r * h_n)
            h = (1.0 - z) * n + z * h
        x = jnp.dot(h.astype(jnp.bfloat16), w1_ref[...],
                    preferred_element_type=jnp.float32) + b1_ref[...]
        x = x * jax.lax.rsqrt(
            jnp.mean(x * x, axis=-1, keepdims=True) + _EPS) * g1_ref[...]
        x_scr[...] = x.astype(jnp.bfloat16)

    xb = x_scr[...]
    a = jnp.dot(xb, sww1_ref[...], preferred_element_type=jnp.float32)
    b = jnp.dot(xb, sww3_ref[...], preferred_element_type=jnp.float32)
    mid = (jax.nn.silu(a) * b).astype(jnp.bfloat16)
    contrib = jnp.dot(mid, fw_ref[...], preferred_element_type=jnp.float32)

    @pl.when(c == 0)
    def _init_acc():
        acc_scr[...] = contrib

    @pl.when(c > 0)
    def _accum():
        acc_scr[...] += contrib

    @pl.when(c == n_c - 1)
    def _finish():
        y = acc_scr[...] + b2_ref[...]
        y = y * jax.lax.rsqrt(
            jnp.mean(y * y, axis=-1, keepdims=True) + _EPS) * g2_ref[...]
        out_ref[...] = y


def kernel(skill_weights, skill_embeddings, W_ih, W_hh, b_ih, b_hh,
           gp_w1, gp_b1, gp_g1, sw_w1, sw_w3, sw_w2, gp_w2, gp_b2, gp_g2):
    batch, n_slots = skill_weights.shape
    embed = skill_embeddings.shape[1]
    dim = gp_w1.shape[1]
    inner = sw_w1.shape[1]
    goal = gp_w2.shape[1]
    n_b = batch // _B_BLK
    n_c = inner // _C_BLK

    bf = jnp.bfloat16
    emb = skill_embeddings.astype(bf)
    wih_t = W_ih.T.astype(bf)
    whh_t = W_hh.T.astype(bf)
    bih = b_ih.reshape(1, -1)
    bhh = b_hh.reshape(1, -1)
    w1 = gp_w1.astype(bf)
    b1 = gp_b1.reshape(1, -1)
    g1 = gp_g1.reshape(1, -1)
    sw1 = sw_w1.astype(bf)
    sw3 = sw_w3.astype(bf)
    b2 = gp_b2.reshape(1, -1)
    g2 = gp_g2.reshape(1, -1)

    fw = pl.pallas_call(
        _fuse_kern,
        out_shape=jax.ShapeDtypeStruct((inner, goal), jnp.bfloat16),
    )(sw_w2.astype(bf), gp_w2.astype(bf))

    three_h = wih_t.shape[1]

    grid = (n_b, n_c)
    const = lambda i, c: (0, 0)
    out = pl.pallas_call(
        _kern,
        grid=grid,
        in_specs=[
            pl.BlockSpec((_B_BLK, n_slots), lambda i, c: (i, 0)),
            pl.BlockSpec((n_slots, embed), const),
            pl.BlockSpec((embed, three_h), const),
            pl.BlockSpec((_HIDDEN, three_h), const),
            pl.BlockSpec((1, three_h), const),
            pl.BlockSpec((1, three_h), const),
            pl.BlockSpec((_HIDDEN, dim), const),
            pl.BlockSpec((1, dim), const),
            pl.BlockSpec((1, dim), const),
            pl.BlockSpec((dim, _C_BLK), lambda i, c: (0, c)),
            pl.BlockSpec((dim, _C_BLK), lambda i, c: (0, c)),
            pl.BlockSpec((_C_BLK, goal), lambda i, c: (c, 0)),
            pl.BlockSpec((1, goal), const),
            pl.BlockSpec((1, goal), const),
        ],
        out_specs=pl.BlockSpec((_B_BLK, goal), lambda i, c: (i, 0)),
        out_shape=jax.ShapeDtypeStruct((batch, goal), jnp.float32),
        scratch_shapes=[
            pltpu.VMEM((_B_BLK, dim), jnp.bfloat16),
            pltpu.VMEM((_B_BLK, goal), jnp.float32),
        ],
        compiler_params=pltpu.CompilerParams(
            dimension_semantics=("arbitrary", "arbitrary"),
        ),
    )(skill_weights, emb, wih_t, whh_t, bih, bhh,
      w1, b1, g1, sw1, sw3, fw, b2, g2)
    return out


# speedup vs baseline: 1.8189x; 1.0520x over previous
import jax
import jax.numpy as jnp
from jax.experimental import pallas as pl
from jax.experimental.pallas import tpu as pltpu

_B_BLK = 1024
_C_BLK = 1024
_N_COMPOSE = 3
_HIDDEN = 256
_EPS = 1e-6


def _fuse_kern(a_ref, b_ref, out_ref):
    out_ref[...] = jnp.dot(a_ref[...], b_ref[...],
                           preferred_element_type=jnp.float32
                           ).astype(jnp.bfloat16)


def _kern(sw_ref, emb_ref, wih_ref, whh_ref, bih_ref, bhh_ref,
          w1_ref, b1_ref, g1_ref, sww1_ref, sww3_ref, fw_ref,
          b2_ref, g2_ref, out_ref, x_scr, acc_scr):
    c = pl.program_id(1)
    n_c = pl.num_programs(1)

    @pl.when(c == 0)
    def _routing_gru_proj():
        sw = sw_ref[...]
        n_slots = sw.shape[1]
        iota = jax.lax.broadcasted_iota(jnp.int32, sw.shape, 1)
        vals, onehots = [], []
        cur = sw
        for _ in range(_N_COMPOSE):
            m = jnp.max(cur, axis=1, keepdims=True)
            idx = jnp.min(jnp.where(cur == m, iota, n_slots), axis=1,
                          keepdims=True)
            oh = iota == idx
            vals.append(m)
            onehots.append(oh)
            cur = jnp.where(oh, -jnp.inf, cur)
        es = [jnp.exp(v - vals[0]) for v in vals]
        denom = es[0] + es[1] + es[2]
        emb = emb_ref[...]
        h = jnp.zeros((sw.shape[0], _HIDDEN), jnp.float32)
        wih = wih_ref[...]
        whh = whh_ref[...]
        bih = bih_ref[...]
        bhh = bhh_ref[...]
        for t in range(_N_COMPOSE):
            mix = es[t] / denom
            w_oh = jnp.where(onehots[t], mix, 0.0).astype(jnp.bfloat16)
            x_t = jnp.dot(w_oh, emb, preferred_element_type=jnp.float32)
            gi = jnp.dot(x_t.astype(jnp.bfloat16), wih,
                         preferred_element_type=jnp.float32) + bih
            gh = jnp.dot(h.astype(jnp.bfloat16), whh,
                         preferred_element_type=jnp.float32) + bhh
            i_r, i_z, i_n = jnp.split(gi, 3, axis=-1)
            h_r, h_z, h_n = jnp.split(gh, 3, axis=-1)
            r = jax.nn.sigmoid(i_r + h_r)
            z = jax.nn.sigmoid(i_z + h_z)
            n = jnp.tanh(i_n + r * h_n)
            h = (1.0 - z) * n + z * h
        x = jnp.dot(h.astype(jnp.bfloat16), w1_ref[...],
                    preferred_element_type=jnp.float32) + b1_ref[...]
        x = x * jax.lax.rsqrt(
            jnp.mean(x * x, axis=-1, keepdims=True) + _EPS) * g1_ref[...]
        x_scr[...] = x.astype(jnp.bfloat16)

    xb = x_scr[...]
    a = jnp.dot(xb, sww1_ref[...], preferred_element_type=jnp.float32)
    b = jnp.dot(xb, sww3_ref[...], preferred_element_type=jnp.float32)
    mid = (jax.nn.silu(a) * b).astype(jnp.bfloat16)
    contrib = jnp.dot(mid, fw_ref[...], preferred_element_type=jnp.float32)

    @pl.when(c == 0)
    def _init_acc():
        acc_scr[...] = contrib

    @pl.when(c > 0)
    def _accum():
        acc_scr[...] += contrib

    @pl.when(c == n_c - 1)
    def _finish():
        y = acc_scr[...] + b2_ref[...]
        y = y * jax.lax.rsqrt(
            jnp.mean(y * y, axis=-1, keepdims=True) + _EPS) * g2_ref[...]
        out_ref[...] = y


def kernel(skill_weights, skill_embeddings, W_ih, W_hh, b_ih, b_hh,
           gp_w1, gp_b1, gp_g1, sw_w1, sw_w3, sw_w2, gp_w2, gp_b2, gp_g2):
    batch, n_slots = skill_weights.shape
    embed = skill_embeddings.shape[1]
    dim = gp_w1.shape[1]
    inner = sw_w1.shape[1]
    goal = gp_w2.shape[1]
    n_b = batch // _B_BLK
    n_c = inner // _C_BLK

    bf = jnp.bfloat16
    emb = skill_embeddings.astype(bf)
    wih_t = W_ih.T.astype(bf)
    whh_t = W_hh.T.astype(bf)
    bih = b_ih.reshape(1, -1)
    bhh = b_hh.reshape(1, -1)
    w1 = gp_w1.astype(bf)
    b1 = gp_b1.reshape(1, -1)
    g1 = gp_g1.reshape(1, -1)
    sw1 = sw_w1.astype(bf)
    sw3 = sw_w3.astype(bf)
    b2 = gp_b2.reshape(1, -1)
    g2 = gp_g2.reshape(1, -1)

    fw = pl.pallas_call(
        _fuse_kern,
        out_shape=jax.ShapeDtypeStruct((inner, goal), jnp.bfloat16),
    )(sw_w2.astype(bf), gp_w2.astype(bf))

    three_h = wih_t.shape[1]

    grid = (n_b, n_c)
    const = lambda i, c: (0, 0)
    out = pl.pallas_call(
        _kern,
        grid=grid,
        in_specs=[
            pl.BlockSpec((_B_BLK, n_slots), lambda i, c: (i, 0)),
            pl.BlockSpec((n_slots, embed), const),
            pl.BlockSpec((embed, three_h), const),
            pl.BlockSpec((_HIDDEN, three_h), const),
            pl.BlockSpec((1, three_h), const),
            pl.BlockSpec((1, three_h), const),
            pl.BlockSpec((_HIDDEN, dim), const),
            pl.BlockSpec((1, dim), const),
            pl.BlockSpec((1, dim), const),
            pl.BlockSpec((dim, _C_BLK), lambda i, c: (0, c)),
            pl.BlockSpec((dim, _C_BLK), lambda i, c: (0, c)),
            pl.BlockSpec((_C_BLK, goal), lambda i, c: (c, 0)),
            pl.BlockSpec((1, goal), const),
            pl.BlockSpec((1, goal), const),
        ],
        out_specs=pl.BlockSpec((_B_BLK, goal), lambda i, c: (i, 0)),
        out_shape=jax.ShapeDtypeStruct((batch, goal), jnp.float32),
        scratch_shapes=[
            pltpu.VMEM((_B_BLK, dim), jnp.bfloat16),
            pltpu.VMEM((_B_BLK, goal), jnp.float32),
        ],
        compiler_params=pltpu.CompilerParams(
            dimension_semantics=("arbitrary", "arbitrary"),
        ),
    )(skill_weights, emb, wih_t, whh_t, bih, bhh,
      w1, b1, g1, sw1, sw3, fw, b2, g2)
    return out


# speedup vs baseline: 1.8367x; 1.0098x over previous
import jax
import jax.numpy as jnp
from jax.experimental import pallas as pl
from jax.experimental.pallas import tpu as pltpu

_B_BLK = 1024
_C_BLK = 1024
_N_COMPOSE = 3
_HIDDEN = 256
_EPS = 1e-6


def _fuse_kern(a_ref, b_ref, out_ref):
    out_ref[...] = jnp.dot(a_ref[...], b_ref[...],
                           preferred_element_type=jnp.float32
                           ).astype(jnp.bfloat16)


def _kern(sw_ref, emb_ref, wih_ref, whh_ref, bih_ref, bhh_ref,
          w1_ref, b1_ref, g1_ref, sww1_ref, sww3_ref, fw_ref,
          b2_ref, g2_ref, out_ref, x_scr, acc_scr):
    c = pl.program_id(1)
    n_c = pl.num_programs(1)

    @pl.when(c == 0)
    def _routing_gru_proj():
        sw = sw_ref[...]
        n_slots = sw.shape[1]
        iota = jax.lax.broadcasted_iota(jnp.int32, sw.shape, 1)
        vals, onehots = [], []
        cur = sw
        for _ in range(_N_COMPOSE):
            m = jnp.max(cur, axis=1, keepdims=True)
            idx = jnp.min(jnp.where(cur == m, iota, n_slots), axis=1,
                          keepdims=True)
            oh = iota == idx
            vals.append(m)
            onehots.append(oh)
            cur = jnp.where(oh, -jnp.inf, cur)
        es = [jnp.exp(v - vals[0]) for v in vals]
        denom = es[0] + es[1] + es[2]
        emb = emb_ref[...]
        h = jnp.zeros((sw.shape[0], _HIDDEN), jnp.float32)
        whh = whh_ref[...]
        bih = bih_ref[...]
        bhh = bhh_ref[...]
        emb_wih = jnp.dot(emb, wih_ref[...],
                          preferred_element_type=jnp.float32
                          ).astype(jnp.bfloat16)
        for t in range(_N_COMPOSE):
            mix = es[t] / denom
            w_oh = jnp.where(onehots[t], mix, 0.0).astype(jnp.bfloat16)
            gi = jnp.dot(w_oh, emb_wih,
                         preferred_element_type=jnp.float32) + bih
            gh = jnp.dot(h.astype(jnp.bfloat16), whh,
                         preferred_element_type=jnp.float32) + bhh
            i_r, i_z, i_n = jnp.split(gi, 3, axis=-1)
            h_r, h_z, h_n = jnp.split(gh, 3, axis=-1)
            r = jax.nn.sigmoid(i_r + h_r)
            z = jax.nn.sigmoid(i_z + h_z)
            n = jnp.tanh(i_n + r * h_n)
            h = (1.0 - z) * n + z * h
        x = jnp.dot(h.astype(jnp.bfloat16), w1_ref[...],
                    preferred_element_type=jnp.float32) + b1_ref[...]
        x = x * jax.lax.rsqrt(
            jnp.mean(x * x, axis=-1, keepdims=True) + _EPS) * g1_ref[...]
        x_scr[...] = x.astype(jnp.bfloat16)

    xb = x_scr[...]
    a = jnp.dot(xb, sww1_ref[...], preferred_element_type=jnp.float32)
    b = jnp.dot(xb, sww3_ref[...], preferred_element_type=jnp.float32)
    mid = (jax.nn.silu(a) * b).astype(jnp.bfloat16)
    contrib = jnp.dot(mid, fw_ref[...], preferred_element_type=jnp.float32)

    @pl.when(c == 0)
    def _init_acc():
        acc_scr[...] = contrib

    @pl.when(c > 0)
    def _accum():
        acc_scr[...] += contrib

    @pl.when(c == n_c - 1)
    def _finish():
        y = acc_scr[...] + b2_ref[...]
        y = y * jax.lax.rsqrt(
            jnp.mean(y * y, axis=-1, keepdims=True) + _EPS) * g2_ref[...]
        out_ref[...] = y


def kernel(skill_weights, skill_embeddings, W_ih, W_hh, b_ih, b_hh,
           gp_w1, gp_b1, gp_g1, sw_w1, sw_w3, sw_w2, gp_w2, gp_b2, gp_g2):
    batch, n_slots = skill_weights.shape
    embed = skill_embeddings.shape[1]
    dim = gp_w1.shape[1]
    inner = sw_w1.shape[1]
    goal = gp_w2.shape[1]
    n_b = batch // _B_BLK
    n_c = inner // _C_BLK

    bf = jnp.bfloat16
    emb = skill_embeddings.astype(bf)
    wih_t = W_ih.T.astype(bf)
    whh_t = W_hh.T.astype(bf)
    bih = b_ih.reshape(1, -1)
    bhh = b_hh.reshape(1, -1)
    w1 = gp_w1.astype(bf)
    b1 = gp_b1.reshape(1, -1)
    g1 = gp_g1.reshape(1, -1)
    sw1 = sw_w1.astype(bf)
    sw3 = sw_w3.astype(bf)
    b2 = gp_b2.reshape(1, -1)
    g2 = gp_g2.reshape(1, -1)

    fw = pl.pallas_call(
        _fuse_kern,
        out_shape=jax.ShapeDtypeStruct((inner, goal), jnp.bfloat16),
    )(sw_w2.astype(bf), gp_w2.astype(bf))

    three_h = wih_t.shape[1]

    grid = (n_b, n_c)
    const = lambda i, c: (0, 0)
    out = pl.pallas_call(
        _kern,
        grid=grid,
        in_specs=[
            pl.BlockSpec((_B_BLK, n_slots), lambda i, c: (i, 0)),
            pl.BlockSpec((n_slots, embed), const),
            pl.BlockSpec((embed, three_h), const),
            pl.BlockSpec((_HIDDEN, three_h), const),
            pl.BlockSpec((1, three_h), const),
            pl.BlockSpec((1, three_h), const),
            pl.BlockSpec((_HIDDEN, dim), const),
            pl.BlockSpec((1, dim), const),
            pl.BlockSpec((1, dim), const),
            pl.BlockSpec((dim, _C_BLK), lambda i, c: (0, c)),
            pl.BlockSpec((dim, _C_BLK), lambda i, c: (0, c)),
            pl.BlockSpec((_C_BLK, goal), lambda i, c: (c, 0)),
            pl.BlockSpec((1, goal), const),
            pl.BlockSpec((1, goal), const),
        ],
        out_specs=pl.BlockSpec((_B_BLK, goal), lambda i, c: (i, 0)),
        out_shape=jax.ShapeDtypeStruct((batch, goal), jnp.float32),
        scratch_shapes=[
            pltpu.VMEM((_B_BLK, dim), jnp.bfloat16),
            pltpu.VMEM((_B_BLK, goal), jnp.float32),
        ],
        compiler_params=pltpu.CompilerParams(
            dimension_semantics=("arbitrary", "arbitrary"),
        ),
    )(skill_weights, emb, wih_t, whh_t, bih, bhh,
      w1, b1, g1, sw1, sw3, fw, b2, g2)
    return out
